# Initial kernel scaffold; baseline (speedup 1.0000x reference)
import jax
import jax.numpy as jnp
from jax.experimental import pallas as pl


def kernel(x, edge_attr, We, be, W1, b1, g1, bt1, W2, b2, g2, bt2, W3, b3, g3, bt3, W4, b4, g4, bt4, W5, b5, g5, bt5, edge_index):
    raise NotImplementedError("write your pallas kernel here")


# speedup vs baseline: 5.2243x; 5.2243x over previous
import functools

import jax
import jax.numpy as jnp
from jax import lax
from jax.experimental import pallas as pl
from jax.experimental.pallas import tpu as pltpu
from jax.experimental.pallas import tpu_sc as plsc

N = 10000
E = 320000
D = 128

NC = 2
NS = 16
NW = NC * NS
EPT = E // NW
K = 80
NCH = EPT // K
NSC = 5
CPS = NCH // NSC
NP = 10240
RPT = NP // NS

_mesh = plsc.VectorSubcoreMesh(core_axis_name="c", subcore_axis_name="s")


@functools.partial(
    pl.kernel,
    out_type=jax.ShapeDtypeStruct((NC * NP, D), jnp.float32),
    mesh=_mesh,
    scratch_types=[
        pltpu.VMEM((CPS, K), jnp.int32),
        pltpu.VMEM((CPS, K), jnp.int32),
        pltpu.VMEM((CPS, K), jnp.float32),
        pltpu.VMEM((2, D), jnp.float32),
        pltpu.VMEM((K, D), jnp.float32),
        pltpu.VMEM_SHARED((NP, D), jnp.float32),
        pltpu.SemaphoreType.DMA,
    ],
)
def _gine_aggr(h_hbm, src_hbm, dst_hbm, a_hbm, wb_hbm, out_hbm,
               src_v, dst_v, a_v, wb_v, rows, acc, sem):
    c = lax.axis_index("c")
    s = lax.axis_index("s")
    wid = c * NS + s

    pltpu.sync_copy(wb_hbm, wb_v)

    zero = jnp.zeros((16,), jnp.float32)

    def zrow(r, carry):
        for seg in range(8):
            rows[r, pl.ds(seg * 16, 16)] = zero
        return carry

    lax.fori_loop(0, K, zrow, 0)
    base_r = s * RPT
    for i in range(RPT // K):
        pltpu.sync_copy(rows, acc.at[pl.ds(base_r + i * K, K)])
    plsc.subcore_barrier()

    w_segs = [wb_v[0, pl.ds(seg * 16, 16)] for seg in range(8)]
    b_segs = [wb_v[1, pl.ds(seg * 16, 16)] for seg in range(8)]

    def superchunk(sc, carry0):
        pltpu.sync_copy(src_hbm.at[wid, sc], src_v)
        pltpu.sync_copy(dst_hbm.at[wid, sc], dst_v)
        pltpu.sync_copy(a_hbm.at[wid, sc], a_v)

        def chunk(ci, carry):
            pltpu.async_copy(h_hbm.at[src_v.at[ci]], rows, sem).wait()

            def group(g, inner):
                av16 = a_v[ci, pl.ds(g * 16, 16)]
                for jj in range(16):
                    av = av16[jj]
                    for seg in range(8):
                        sl = pl.ds(seg * 16, 16)
                        rows[g * 16 + jj, sl] = jnp.maximum(
                            rows[g * 16 + jj, sl]
                            + (av * w_segs[seg] + b_segs[seg]), 0.0)
                return inner

            lax.fori_loop(0, K // 16, group, 0)
            pltpu.sync_copy(rows, acc.at[dst_v.at[ci]], add=True)
            return carry

        lax.fori_loop(0, CPS, chunk, 0)
        return carry0

    lax.fori_loop(0, NSC, superchunk, 0)
    plsc.subcore_barrier()

    pltpu.sync_copy(acc.at[pl.ds(base_r, RPT)],
                    out_hbm.at[pl.ds(c * NP + base_r, RPT)])


def _mlp_bn(h, W, b, g, bt):
    h = jnp.maximum(jnp.dot(h, W, preferred_element_type=jnp.float32) + b, 0.0)
    mu = jnp.mean(h, axis=0, keepdims=True)
    xc = h - mu
    var = jnp.mean(xc * xc, axis=0, keepdims=True)
    return xc * lax.rsqrt(var + 1e-5) * g + bt


def _tc1_body(x_ref, p_ref, W_ref, b_ref, g_ref, bt_ref, o_ref):
    p = p_ref[...]
    y = x_ref[...] + p[:N] + p[NP:NP + N]
    o_ref[...] = _mlp_bn(y, W_ref[...], b_ref[...], g_ref[...], bt_ref[...])


def _tc2_body(x1_ref, p_ref, W2r, b2r, g2r, bt2r, W3r, b3r, g3r, bt3r,
              W4r, b4r, g4r, bt4r, W5r, b5r, g5r, bt5r, o_ref):
    x1 = x1_ref[...]
    p = p_ref[...]
    y2 = x1 + p[:N] + p[NP:NP + N]
    x2 = _mlp_bn(y2, W2r[...], b2r[...], g2r[...], bt2r[...])
    h = jnp.concatenate([x1, x2], axis=1)
    h = _mlp_bn(h, W3r[...], b3r[...], g3r[...], bt3r[...])
    h = _mlp_bn(h, W4r[...], b4r[...], g4r[...], bt4r[...])
    o_ref[...] = _mlp_bn(h, W5r[...], b5r[...], g5r[...], bt5r[...])


_tc1 = pl.pallas_call(
    _tc1_body, out_shape=jax.ShapeDtypeStruct((N, D), jnp.float32))
_tc2 = pl.pallas_call(
    _tc2_body, out_shape=jax.ShapeDtypeStruct((N, 16), jnp.float32))


def kernel(x, edge_attr, We, be, W1, b1, g1, bt1, W2, b2, g2, bt2,
           W3, b3, g3, bt3, W4, b4, g4, bt4, W5, b5, g5, bt5, edge_index):
    src3 = edge_index[0].reshape(NW, NSC, CPS, K)
    dst3 = edge_index[1].reshape(NW, NSC, CPS, K)
    a3 = edge_attr.reshape(NW, NSC, CPS, K)
    wb = jnp.concatenate([We, be.reshape(1, D)], axis=0)

    r = lambda v: v.reshape(1, -1)
    p1 = _gine_aggr(x, src3, dst3, a3, wb)
    x1 = _tc1(x, p1, W1, r(b1), r(g1), r(bt1))
    p2 = _gine_aggr(x1, src3, dst3, a3, wb)
    out = _tc2(x1, p2, W2, r(b2), r(g2), r(bt2), W3, r(b3), r(g3), r(bt3),
               W4, r(b4), r(g4), r(bt4), W5, r(b5), r(g5), r(bt5))
    return out


# speedup vs baseline: 7.1102x; 1.3610x over previous
import functools

import jax
import jax.numpy as jnp
from jax import lax
from jax.experimental import pallas as pl
from jax.experimental.pallas import tpu as pltpu
from jax.experimental.pallas import tpu_sc as plsc

N = 10000
E = 320000
D = 128

NC = 2
NS = 16
NW = NC * NS
EPT = E // NW
K = 80
NCH = EPT // K
NSC = 5
CPS = NCH // NSC
NP = 10240
RPT = NP // NS

_mesh = plsc.VectorSubcoreMesh(core_axis_name="c", subcore_axis_name="s")


@functools.partial(
    pl.kernel,
    out_type=jax.ShapeDtypeStruct((NC * NP, D), jnp.float32),
    mesh=_mesh,
    scratch_types=[
        pltpu.VMEM((CPS, K), jnp.int32),
        pltpu.VMEM((CPS, K), jnp.int32),
        pltpu.VMEM((CPS, K), jnp.float32),
        pltpu.VMEM((2, D), jnp.float32),
        pltpu.VMEM((K, D), jnp.float32),
        pltpu.VMEM((K, D), jnp.float32),
        pltpu.VMEM((K, D), jnp.float32),
        pltpu.SemaphoreType.DMA,
        pltpu.SemaphoreType.DMA,
        pltpu.SemaphoreType.DMA,
        pltpu.SemaphoreType.DMA,
        pltpu.SemaphoreType.DMA,
        pltpu.SemaphoreType.DMA,
        pltpu.VMEM_SHARED((NP, D), jnp.float32),
    ],
)
def _gine_aggr(h_hbm, src_hbm, dst_hbm, a_hbm, wb_hbm, out_hbm,
               src_v, dst_v, a_v, wb_v, r0, r1, r2,
               g0, g1, g2, s0, s1, s2, acc):
    c = lax.axis_index("c")
    s = lax.axis_index("s")
    wid = c * NS + s

    pltpu.sync_copy(wb_hbm, wb_v)

    zero = jnp.zeros((16,), jnp.float32)

    def zrow(r, carry):
        for seg in range(8):
            r0[r, pl.ds(seg * 16, 16)] = zero
        return carry

    lax.fori_loop(0, K, zrow, 0)
    base_r = s * RPT
    for i in range(RPT // K):
        pltpu.sync_copy(r0, acc.at[pl.ds(base_r + i * K, K)])
    plsc.subcore_barrier()

    w_segs = [wb_v[0, pl.ds(seg * 16, 16)] for seg in range(8)]
    b_segs = [wb_v[1, pl.ds(seg * 16, 16)] for seg in range(8)]

    bufs = (r0, r1, r2)
    gsem = (g0, g1, g2)
    ssem = (s0, s1, s2)

    def gather_start(ci, b):
        pltpu.async_copy(h_hbm.at[src_v.at[ci]], bufs[b], gsem[b])

    def gather_wait(ci, b):
        pltpu.make_async_copy(h_hbm.at[src_v.at[ci]], bufs[b], gsem[b]).wait()

    def scat_start(ci, b):
        pltpu.async_copy(bufs[b], acc.at[dst_v.at[ci]], ssem[b], add=True)

    def scat_wait(ci, b):
        pltpu.make_async_copy(bufs[b], acc.at[dst_v.at[ci]], ssem[b]).wait()

    def compute(ci, b):
        buf = bufs[b]

        def group(g, inner):
            av16 = a_v[ci, pl.ds(g * 16, 16)]
            for jj in range(16):
                av = av16[jj]
                for seg in range(8):
                    sl = pl.ds(seg * 16, 16)
                    buf[g * 16 + jj, sl] = jnp.maximum(
                        buf[g * 16 + jj, sl]
                        + (av * w_segs[seg] + b_segs[seg]), 0.0)
            return inner

        lax.fori_loop(0, K // 16, group, 0)

    def superchunk(sc, carry0):
        pltpu.sync_copy(src_hbm.at[wid, sc], src_v)
        pltpu.sync_copy(dst_hbm.at[wid, sc], dst_v)
        pltpu.sync_copy(a_hbm.at[wid, sc], a_v)

        gather_start(0, 0)
        gather_start(1, 1)
        gather_wait(0, 0)
        compute(0, 0)
        scat_start(0, 0)
        gather_start(2, 2)
        gather_wait(1, 1)
        compute(1, 1)
        scat_start(1, 1)
        scat_wait(0, 0)
        gather_start(3, 0)

        def triple(i, carry):
            for b in (2, 0, 1):
                c = 3 * i + (b - 2) % 3 + 2
                gather_wait(c, b)
                compute(c, b)
                scat_start(c, b)
                b2 = (b + 2) % 3
                scat_wait(c - 1, b2)
                gather_start(c + 2, b2)
            return carry

        lax.fori_loop(0, 7, triple, 0)

        gather_wait(23, 2)
        compute(23, 2)
        scat_start(23, 2)
        gather_wait(24, 0)
        compute(24, 0)
        scat_start(24, 0)
        scat_wait(22, 1)
        scat_wait(23, 2)
        scat_wait(24, 0)
        return carry0

    lax.fori_loop(0, NSC, superchunk, 0)
    plsc.subcore_barrier()

    pltpu.sync_copy(acc.at[pl.ds(base_r, RPT)],
                    out_hbm.at[pl.ds(c * NP + base_r, RPT)])


def _mlp_bn(h, W, b, g, bt):
    h = jnp.maximum(jnp.dot(h, W, preferred_element_type=jnp.float32) + b, 0.0)
    mu = jnp.mean(h, axis=0, keepdims=True)
    xc = h - mu
    var = jnp.mean(xc * xc, axis=0, keepdims=True)
    return xc * lax.rsqrt(var + 1e-5) * g + bt


def _tc1_body(x_ref, p_ref, W_ref, b_ref, g_ref, bt_ref, o_ref):
    p = p_ref[...]
    y = x_ref[...] + p[:N] + p[NP:NP + N]
    o_ref[...] = _mlp_bn(y, W_ref[...], b_ref[...], g_ref[...], bt_ref[...])


def _tc2_body(x1_ref, p_ref, W2r, b2r, g2r, bt2r, W3r, b3r, g3r, bt3r,
              W4r, b4r, g4r, bt4r, W5r, b5r, g5r, bt5r, o_ref):
    x1 = x1_ref[...]
    p = p_ref[...]
    y2 = x1 + p[:N] + p[NP:NP + N]
    x2 = _mlp_bn(y2, W2r[...], b2r[...], g2r[...], bt2r[...])
    h = jnp.concatenate([x1, x2], axis=1)
    h = _mlp_bn(h, W3r[...], b3r[...], g3r[...], bt3r[...])
    h = _mlp_bn(h, W4r[...], b4r[...], g4r[...], bt4r[...])
    o_ref[...] = _mlp_bn(h, W5r[...], b5r[...], g5r[...], bt5r[...])


_tc1 = pl.pallas_call(
    _tc1_body, out_shape=jax.ShapeDtypeStruct((N, D), jnp.float32))
_tc2 = pl.pallas_call(
    _tc2_body, out_shape=jax.ShapeDtypeStruct((N, 16), jnp.float32))


def kernel(x, edge_attr, We, be, W1, b1, g1, bt1, W2, b2, g2, bt2,
           W3, b3, g3, bt3, W4, b4, g4, bt4, W5, b5, g5, bt5, edge_index):
    src3 = edge_index[0].reshape(NW, NSC, CPS, K)
    dst3 = edge_index[1].reshape(NW, NSC, CPS, K)
    a3 = edge_attr.reshape(NW, NSC, CPS, K)
    wb = jnp.concatenate([We, be.reshape(1, D)], axis=0)

    r = lambda v: v.reshape(1, -1)
    p1 = _gine_aggr(x, src3, dst3, a3, wb)
    x1 = _tc1(x, p1, W1, r(b1), r(g1), r(bt1))
    p2 = _gine_aggr(x1, src3, dst3, a3, wb)
    out = _tc2(x1, p2, W2, r(b2), r(g2), r(bt2), W3, r(b3), r(g3), r(bt3),
               W4, r(b4), r(g4), r(bt4), W5, r(b5), r(g5), r(bt5))
    return out
